# baseline (speedup 1.0000x reference)
import functools
import math

import jax
import jax.numpy as jnp
from jax import lax
from jax.experimental import pallas as pl
from jax.experimental.pallas import tpu as pltpu
from jax.experimental.pallas import tpu_sc as plsc

DT_SCALE = math.log1p(168.0)

B, L = 16384, 200
TOT = B * L
NC, NS = 2, 16
NW = NC * NS
PER_W = TOT // NW
CHUNK = 12800
N_CHUNKS = PER_W // CHUNK
VECS = CHUNK // 16

_mesh = plsc.VectorSubcoreMesh(core_axis_name="c", subcore_axis_name="s")


@functools.partial(
    pl.kernel,
    mesh=_mesh,
    compiler_params=pltpu.CompilerParams(needs_layout_passes=False),
    out_type=jax.ShapeDtypeStruct((TOT,), jnp.float32),
    scratch_types=[
        pltpu.VMEM((16,), jnp.float32),
        pltpu.VMEM((16,), jnp.float32),
        pltpu.VMEM((CHUNK,), jnp.float32),
        pltpu.VMEM((CHUNK,), jnp.float32),
        pltpu.VMEM((CHUNK,), jnp.int32),
        pltpu.VMEM((CHUNK,), jnp.float32),
    ],
)
def _sc_recovery(mpc_hbm, dt_hbm, idx_hbm, ltau_hbm, out_hbm,
                 ltau_v, itau_v, mpc_v, dt_v, idx_v, out_v):
    wid = lax.axis_index("s") * NC + lax.axis_index("c")
    base = wid * PER_W

    pltpu.sync_copy(ltau_hbm, ltau_v)
    itau_v[...] = jnp.exp(-ltau_v[...])

    def chunk_body(c, carry):
        off = base + c * CHUNK
        pltpu.sync_copy(mpc_hbm.at[pl.ds(off, CHUNK)], mpc_v)
        pltpu.sync_copy(dt_hbm.at[pl.ds(off, CHUNK)], dt_v)
        pltpu.sync_copy(idx_hbm.at[pl.ds(off, CHUNK)], idx_v)

        def vec_body(i, inner):
            s = pl.ds(i * 16, 16)
            inv_tau = plsc.load_gather(itau_v, [idx_v[s]])
            dt_hours = jnp.exp(dt_v[s] * DT_SCALE) - 1.0
            out_v[s] = 1.0 - (1.0 - mpc_v[s]) * jnp.exp(-dt_hours * inv_tau)
            return inner

        lax.fori_loop(0, VECS, vec_body, 0)
        pltpu.sync_copy(out_v, out_hbm.at[pl.ds(off, CHUNK)])
        return carry

    lax.fori_loop(0, N_CHUNKS, chunk_body, 0)


def kernel(mpc, delta_t, muscle_idx, log_tau):
    ltau16 = jnp.zeros((16,), jnp.float32).at[:15].set(log_tau)
    out = _sc_recovery(
        mpc.reshape(-1),
        delta_t.reshape(-1),
        muscle_idx.reshape(-1).astype(jnp.int32),
        ltau16,
    )
    return out.reshape(mpc.shape)


# speedup vs baseline: 1.4803x; 1.4803x over previous
import functools
import math

import jax
import jax.numpy as jnp
from jax import lax
from jax.experimental import pallas as pl
from jax.experimental.pallas import tpu as pltpu
from jax.experimental.pallas import tpu_sc as plsc

DT_SCALE = math.log1p(168.0)

B, L = 16384, 200
TOT = B * L
NC, NS = 2, 16
NW = NC * NS
PER_W = TOT // NW
CHUNK = 10240
N_CHUNKS = PER_W // CHUNK

_mesh = plsc.VectorSubcoreMesh(core_axis_name="c", subcore_axis_name="s")


@functools.partial(
    pl.kernel,
    mesh=_mesh,
    compiler_params=pltpu.CompilerParams(needs_layout_passes=False),
    out_type=jax.ShapeDtypeStruct((TOT,), jnp.float32),
    scratch_types=[
        pltpu.VMEM((16,), jnp.float32),
        pltpu.VMEM((16,), jnp.float32),
        pltpu.VMEM((2, CHUNK), jnp.float32),
        pltpu.VMEM((2, CHUNK), jnp.float32),
        pltpu.VMEM((2, CHUNK), jnp.int32),
        pltpu.VMEM((2, CHUNK), jnp.float32),
        pltpu.SemaphoreType.DMA((2,)),
        pltpu.SemaphoreType.DMA((2,)),
    ],
)
def _sc_recovery(mpc_hbm, dt_hbm, idx_hbm, ltau_hbm, out_hbm,
                 ltau_v, itau_v, mpc_v, dt_v, idx_v, out_v,
                 in_sem, out_sem):
    wid = lax.axis_index("s") * NC + lax.axis_index("c")
    base = wid * PER_W

    pltpu.sync_copy(ltau_hbm, ltau_v)
    itau_v[...] = -jnp.exp(-ltau_v[...])

    def in_copies(c, b):
        off = base + c * CHUNK
        sl = pl.ds(off, CHUNK)
        return (
            pltpu.make_async_copy(mpc_hbm.at[sl], mpc_v.at[b], in_sem.at[b]),
            pltpu.make_async_copy(dt_hbm.at[sl], dt_v.at[b], in_sem.at[b]),
            pltpu.make_async_copy(idx_hbm.at[sl], idx_v.at[b], in_sem.at[b]),
        )

    def out_copy(c, b):
        off = base + c * CHUNK
        return pltpu.make_async_copy(
            out_v.at[b], out_hbm.at[pl.ds(off, CHUNK)], out_sem.at[b])

    def start_in(c):
        for cp in in_copies(c, c % 2):
            cp.start()

    def wait_in(c):
        for cp in in_copies(c, c % 2):
            cp.wait()

    start_in(0)
    start_in(1)
    for c in range(N_CHUNKS):
        b = c % 2
        wait_in(c)
        if c >= 2:
            out_copy(c - 2, b).wait()

        @plsc.parallel_loop(0, CHUNK, step=16, unroll=8)
        def _compute(i, _b=b):
            s = pl.ds(i, 16)
            neg_inv_tau = plsc.load_gather(itau_v, [idx_v[_b, s]])
            e1 = jnp.exp(dt_v[_b, s] * DT_SCALE)
            out_v[_b, s] = 1.0 - (1.0 - mpc_v[_b, s]) * jnp.exp(
                (e1 - 1.0) * neg_inv_tau)

        out_copy(c, b).start()
        if c + 2 < N_CHUNKS:
            start_in(c + 2)
    out_copy(N_CHUNKS - 2, N_CHUNKS % 2).wait()
    out_copy(N_CHUNKS - 1, (N_CHUNKS - 1) % 2).wait()


def kernel(mpc, delta_t, muscle_idx, log_tau):
    ltau16 = jnp.zeros((16,), jnp.float32).at[:15].set(log_tau)
    out = _sc_recovery(
        mpc.reshape(-1),
        delta_t.reshape(-1),
        muscle_idx.reshape(-1).astype(jnp.int32),
        ltau16,
    )
    return out.reshape(mpc.shape)


# speedup vs baseline: 2.5360x; 1.7131x over previous
import functools
import math

import jax
import jax.numpy as jnp
from jax import lax
from jax.experimental import pallas as pl
from jax.experimental.pallas import tpu as pltpu
from jax.experimental.pallas import tpu_sc as plsc

DT_SCALE = math.log1p(168.0)

B, L = 16384, 200
NC, NS = 2, 16
NW = NC * NS
ROWS_W = B // NW
RBLK = 32
N_CHUNKS = ROWS_W // RBLK
COLS = list(range(0, L - 16 + 1, 16)) + [L - 16]

_mesh = plsc.VectorSubcoreMesh(core_axis_name="c", subcore_axis_name="s")


@functools.partial(
    pl.kernel,
    mesh=_mesh,
    compiler_params=pltpu.CompilerParams(needs_layout_passes=False),
    out_type=jax.ShapeDtypeStruct((B, L), jnp.float32),
    scratch_types=[
        pltpu.VMEM((16,), jnp.float32),
        pltpu.VMEM((16,), jnp.float32),
        pltpu.VMEM((2, RBLK, L), jnp.float32),
        pltpu.VMEM((2, RBLK, L), jnp.float32),
        pltpu.VMEM((2, RBLK, L), jnp.int32),
        pltpu.VMEM((2, RBLK, L), jnp.float32),
        pltpu.SemaphoreType.DMA((2,)),
        pltpu.SemaphoreType.DMA((2,)),
    ],
)
def _sc_recovery(mpc_hbm, dt_hbm, idx_hbm, ltau_hbm, out_hbm,
                 ltau_v, itau_v, mpc_v, dt_v, idx_v, out_v,
                 in_sem, out_sem):
    wid = lax.axis_index("s") * NC + lax.axis_index("c")
    base = wid * ROWS_W

    pltpu.sync_copy(ltau_hbm, ltau_v)
    itau_v[...] = -jnp.exp(-ltau_v[...])

    def in_copies(c, b):
        sl = pl.ds(base + c * RBLK, RBLK)
        return (
            pltpu.make_async_copy(mpc_hbm.at[sl], mpc_v.at[b], in_sem.at[b]),
            pltpu.make_async_copy(dt_hbm.at[sl], dt_v.at[b], in_sem.at[b]),
            pltpu.make_async_copy(idx_hbm.at[sl], idx_v.at[b], in_sem.at[b]),
        )

    def out_copy(c, b):
        sl = pl.ds(base + c * RBLK, RBLK)
        return pltpu.make_async_copy(out_v.at[b], out_hbm.at[sl],
                                     out_sem.at[b])

    def start_in(c):
        for cp in in_copies(c, c % 2):
            cp.start()

    def wait_in(c):
        for cp in in_copies(c, c % 2):
            cp.wait()

    def compute(b):
        @plsc.parallel_loop(0, RBLK, step=1)
        def _compute(r):
            for col in COLS:
                s = pl.ds(col, 16)
                neg_inv_tau = plsc.load_gather(itau_v, [idx_v[b, r, s]])
                e1 = jnp.exp(dt_v[b, r, s] * DT_SCALE)
                out_v[b, r, s] = 1.0 - (1.0 - mpc_v[b, r, s]) * jnp.exp(
                    (e1 - 1.0) * neg_inv_tau)

    start_in(0)
    start_in(1)
    for c in (0, 1):
        wait_in(c)
        compute(c % 2)
        out_copy(c, c % 2).start()
        start_in(c + 2)

    def pair_body(c2, carry):
        for b in (0, 1):
            c = c2 * 2 + b
            wait_in(c)
            out_copy(c - 2, b).wait()
            compute(b)
            out_copy(c, b).start()
            start_in(c + 2)
        return carry

    lax.fori_loop(1, N_CHUNKS // 2 - 1, pair_body, 0)

    for c in (N_CHUNKS - 2, N_CHUNKS - 1):
        b = c % 2
        wait_in(c)
        out_copy(c - 2, b).wait()
        compute(b)
        out_copy(c, b).start()
    out_copy(N_CHUNKS - 2, N_CHUNKS % 2).wait()
    out_copy(N_CHUNKS - 1, (N_CHUNKS - 1) % 2).wait()


def kernel(mpc, delta_t, muscle_idx, log_tau):
    ltau16 = jnp.zeros((16,), jnp.float32).at[:15].set(log_tau)
    return _sc_recovery(mpc, delta_t, muscle_idx.astype(jnp.int32), ltau16)


# speedup vs baseline: 5.7975x; 2.2860x over previous
import functools
import math

import jax
import jax.numpy as jnp
from jax import lax
from jax.experimental import pallas as pl
from jax.experimental.pallas import tpu as pltpu
from jax.experimental.pallas import tpu_sc as plsc

DT_SCALE = math.log1p(168.0)

B, L = 16384, 200
NC, NS = 2, 16
NW = NC * NS
CW = B // NW
RBLK = 8
N_CHUNKS = L // RBLK
CHUNK = RBLK * CW

_mesh = plsc.VectorSubcoreMesh(core_axis_name="c", subcore_axis_name="s")


@functools.partial(
    pl.kernel,
    mesh=_mesh,
    compiler_params=pltpu.CompilerParams(needs_layout_passes=False),
    out_type=jax.ShapeDtypeStruct((L, B), jnp.float32),
    scratch_types=[
        pltpu.VMEM((16,), jnp.float32),
        pltpu.VMEM((16,), jnp.float32),
        pltpu.VMEM((2, RBLK, CW), jnp.float32),
        pltpu.VMEM((2, RBLK, CW), jnp.float32),
        pltpu.VMEM((2, RBLK, CW), jnp.int32),
        pltpu.VMEM((2, RBLK, CW), jnp.float32),
        pltpu.SemaphoreType.DMA((2,)),
        pltpu.SemaphoreType.DMA((2,)),
    ],
)
def _sc_recovery(mpc_hbm, dt_hbm, idx_hbm, ltau_hbm, out_hbm,
                 ltau_v, itau_v, mpc_v, dt_v, idx_v, out_v,
                 in_sem, out_sem):
    wid = lax.axis_index("s") * NC + lax.axis_index("c")
    col0 = wid * CW

    pltpu.sync_copy(ltau_hbm, ltau_v)
    itau_v[...] = -jnp.exp(-ltau_v[...])

    def in_copies(c, b):
        rs = pl.ds(c * RBLK, RBLK)
        cs = pl.ds(col0, CW)
        return (
            pltpu.make_async_copy(mpc_hbm.at[rs, cs], mpc_v.at[b], in_sem.at[b]),
            pltpu.make_async_copy(dt_hbm.at[rs, cs], dt_v.at[b], in_sem.at[b]),
            pltpu.make_async_copy(idx_hbm.at[rs, cs], idx_v.at[b], in_sem.at[b]),
        )

    def out_copy(c, b):
        return pltpu.make_async_copy(
            out_v.at[b], out_hbm.at[pl.ds(c * RBLK, RBLK), pl.ds(col0, CW)],
            out_sem.at[b])

    def start_in(c):
        for cp in in_copies(c, c % 2):
            cp.start()

    def wait_in(c):
        for cp in in_copies(c, c % 2):
            cp.wait()

    def compute(b):
        @plsc.parallel_loop(0, RBLK * CW, step=16, unroll=8)
        def _compute(i):
            r = i >> 9
            col = i & (CW - 1)
            s = pl.ds(col, 16)
            neg_inv_tau = plsc.load_gather(itau_v, [idx_v[b, r, s]])
            e1 = jnp.exp(dt_v[b, r, s] * DT_SCALE)
            out_v[b, r, s] = 1.0 - (1.0 - mpc_v[b, r, s]) * jnp.exp(
                (e1 - 1.0) * neg_inv_tau)

    start_in(0)
    start_in(1)
    for c in (0, 1):
        wait_in(c)
        compute(c % 2)
        out_copy(c, c % 2).start()
        start_in(c + 2)

    def pair_body(c2, carry):
        for b in (0, 1):
            c = c2 * 2 + b
            wait_in(c)
            out_copy(c - 2, b).wait()
            compute(b)
            out_copy(c, b).start()
            start_in(c + 2)
        return carry

    lax.fori_loop(1, N_CHUNKS // 2 - 1, pair_body, 0)

    for c in (N_CHUNKS - 3, N_CHUNKS - 2, N_CHUNKS - 1):
        b = c % 2
        wait_in(c)
        out_copy(c - 2, b).wait()
        compute(b)
        out_copy(c, b).start()
        if c + 2 < N_CHUNKS:
            start_in(c + 2)
    out_copy(N_CHUNKS - 2, (N_CHUNKS - 2) % 2).wait()
    out_copy(N_CHUNKS - 1, (N_CHUNKS - 1) % 2).wait()


def kernel(mpc, delta_t, muscle_idx, log_tau):
    ltau16 = jnp.zeros((16,), jnp.float32).at[:15].set(log_tau)
    out_t = _sc_recovery(mpc.T, delta_t.T, muscle_idx.astype(jnp.int32).T,
                         ltau16)
    return out_t.T


# speedup vs baseline: 6.9022x; 1.1906x over previous
import functools
import math

import jax
import jax.numpy as jnp
from jax import lax
from jax.experimental import pallas as pl
from jax.experimental.pallas import tpu as pltpu
from jax.experimental.pallas import tpu_sc as plsc

DT_SCALE = math.log1p(168.0)

B, L = 16384, 200
NC, NS = 2, 16
NW = NC * NS
CW = B // NW
RBLK = 8
N_CHUNKS = L // RBLK
NBUF = 4

_mesh = plsc.VectorSubcoreMesh(core_axis_name="c", subcore_axis_name="s")


@functools.partial(
    pl.kernel,
    mesh=_mesh,
    compiler_params=pltpu.CompilerParams(needs_layout_passes=False),
    out_type=jax.ShapeDtypeStruct((L, B), jnp.float32),
    scratch_types=[
        pltpu.VMEM((16,), jnp.float32),
        pltpu.VMEM((16,), jnp.float32),
        pltpu.VMEM((NBUF, RBLK, CW), jnp.float32),
        pltpu.VMEM((NBUF, RBLK, CW), jnp.float32),
        pltpu.VMEM((NBUF, RBLK, CW), jnp.int32),
        pltpu.VMEM((NBUF, RBLK, CW), jnp.float32),
        pltpu.SemaphoreType.DMA((NBUF,)),
        pltpu.SemaphoreType.DMA((NBUF,)),
    ],
)
def _sc_recovery(mpc_hbm, dt_hbm, idx_hbm, ltau_hbm, out_hbm,
                 ltau_v, itau_v, mpc_v, dt_v, idx_v, out_v,
                 in_sem, out_sem):
    wid = lax.axis_index("s") * NC + lax.axis_index("c")
    col0 = wid * CW

    pltpu.sync_copy(ltau_hbm, ltau_v)
    itau_v[...] = -jnp.exp(-ltau_v[...])

    def in_copies(c, b):
        rs = pl.ds(c * RBLK, RBLK)
        cs = pl.ds(col0, CW)
        return (
            pltpu.make_async_copy(mpc_hbm.at[rs, cs], mpc_v.at[b], in_sem.at[b]),
            pltpu.make_async_copy(dt_hbm.at[rs, cs], dt_v.at[b], in_sem.at[b]),
            pltpu.make_async_copy(idx_hbm.at[rs, cs], idx_v.at[b], in_sem.at[b]),
        )

    def out_copy(c, b):
        return pltpu.make_async_copy(
            out_v.at[b], out_hbm.at[pl.ds(c * RBLK, RBLK), pl.ds(col0, CW)],
            out_sem.at[b])

    def start_in(c, b):
        for cp in in_copies(c, b):
            cp.start()

    for c in range(NBUF):
        start_in(c, c)

    def chunk_body(c, carry):
        b = c & (NBUF - 1)
        for cp in in_copies(c, b):
            cp.wait()

        @pl.when(c >= NBUF)
        def _drain():
            out_copy(c - NBUF, b).wait()

        @plsc.parallel_loop(0, RBLK * CW, step=16, unroll=8)
        def _compute(i):
            r = i >> 9
            col = i & (CW - 1)
            s = pl.ds(col, 16)
            neg_inv_tau = plsc.load_gather(itau_v, [idx_v[b, r, s]])
            e1 = jnp.exp(dt_v[b, r, s] * DT_SCALE)
            out_v[b, r, s] = 1.0 - (1.0 - mpc_v[b, r, s]) * jnp.exp(
                (e1 - 1.0) * neg_inv_tau)

        out_copy(c, b).start()

        @pl.when(c + NBUF < N_CHUNKS)
        def _prefetch():
            start_in(c + NBUF, b)

        return carry

    lax.fori_loop(0, N_CHUNKS, chunk_body, 0)

    for c in range(N_CHUNKS - NBUF, N_CHUNKS):
        out_copy(c, c & (NBUF - 1)).wait()


def kernel(mpc, delta_t, muscle_idx, log_tau):
    ltau16 = jnp.zeros((16,), jnp.float32).at[:15].set(log_tau)
    out_t = _sc_recovery(mpc.T, delta_t.T, muscle_idx.astype(jnp.int32).T,
                         ltau16)
    return out_t.T
